# baseline (speedup 1.0000x reference)
import jax
import jax.numpy as jnp
from jax import lax
from jax.experimental import pallas as pl
from jax.experimental.pallas import tpu as pltpu

_TB = 1024
_CB = 2000


def _softplus(x):
    return jnp.logaddexp(x, 0.0)


def _itaug_build_kernel(emb_ref, g_ref, out_ref):
    CB, _, D = emb_ref.shape
    out_ref[:, :, :D] = emb_ref[...]
    out_ref[:, :, D:D + 1] = g_ref[...].reshape(CB, 1, 1)
    out_ref[:, :, D + 1:] = jnp.zeros((CB, 1, D - 1), jnp.float32)


def _user_gather_kernel(ids_ref, tab_ref, out_ref):
    for mi in range(out_ref.shape[0]):
        out_ref[mi] = tab_ref[ids_ref[0, 0, mi], 0]


def _item_compute_kernel(ids_ref, tab_ref, u_ref, out_ref, vi_s, vj_s):
    TB, D = u_ref.shape
    K = 3 * D + 4

    for mi in range(TB):
        vi_s[mi] = tab_ref[ids_ref[0, 0, mi], 0]
        vj_s[mi] = tab_ref[ids_ref[0, 1, mi], 0]

    u = u_ref[...]
    vi = vi_s[:, :D]
    vj = vj_s[:, :D]
    gi = vi_s[:, D:D + 1]
    gj = vj_s[:, D:D + 1]

    gi_h = gi.astype(jnp.bfloat16).astype(jnp.float32)
    gj_h = gj.astype(jnp.bfloat16).astype(jnp.float32)
    gi_l = gi - gi_h
    gj_l = gj - gj_h

    P = jnp.concatenate(
        [u * vi, u * vj, u * u + vi * vi + vj * vj,
         gi_h, gj_h, gi_l, gj_l], axis=1)
    rowi = lax.broadcasted_iota(jnp.int32, (8, K), 0)
    coli = lax.broadcasted_iota(jnp.int32, (8, K), 1)
    S = (((rowi == coli // D) & (coli < 3 * D))
         | ((coli >= 3 * D) & (rowi == coli - 3 * D + 3))).astype(jnp.float32)
    R = lax.dot_general(S, P, (((1,), (1,)), ((), ())),
                        preferred_element_type=jnp.float32)

    gate = R[3:5, :] + R[5:7, :]
    pred = _softplus(R[0:2, :]) * gate
    out_ref[...] = jnp.concatenate([pred, R[2:3, :]], axis=0)


def kernel(embed_user, embed_item, q, user, item_i, item_j):
    B = int(user.shape[0])
    Nu, D = int(embed_user.shape[0]), int(embed_user.shape[1])
    Ni = int(embed_item.shape[0])

    TB = _TB
    nt = -(-B // TB)
    Bp = nt * TB
    pad = Bp - B

    def pad_ids(x):
        x = x.astype(jnp.int32)
        return jnp.pad(x, (0, pad)) if pad else x

    u_ids = pad_ids(user).reshape(nt, 1, TB)
    ij_ids = jnp.stack([pad_ids(item_i).reshape(nt, TB),
                        pad_ids(item_j).reshape(nt, TB)], axis=1)

    ut = embed_user.astype(jnp.float32).reshape(Nu, 1, D)

    cparams = pltpu.CompilerParams(
        dimension_semantics=("arbitrary",),
        vmem_limit_bytes=57 * 1024 * 1024,
    )
    cparams2 = pltpu.CompilerParams(
        dimension_semantics=("arbitrary",),
        vmem_limit_bytes=63 * 1024 * 1024,
    )

    CB = _CB
    nb = -(-Ni // CB)
    emb3 = embed_item.astype(jnp.float32).reshape(Ni, 1, D)
    g = jnp.tanh(_softplus(q.astype(jnp.float32))).reshape(Ni, 1)
    it_aug = pl.pallas_call(
        _itaug_build_kernel,
        out_shape=jax.ShapeDtypeStruct((Ni, 1, 2 * D), jnp.float32),
        grid=(nb,),
        in_specs=[
            pl.BlockSpec((CB, 1, D), lambda t: (t, 0, 0)),
            pl.BlockSpec((CB, 1), lambda t: (t, 0)),
        ],
        out_specs=pl.BlockSpec((CB, 1, 2 * D), lambda t: (t, 0, 0)),
        compiler_params=pltpu.CompilerParams(
            dimension_semantics=("arbitrary",),
            vmem_limit_bytes=32 * 1024 * 1024,
        ),
    )(emb3, g)

    ug = pl.pallas_call(
        _user_gather_kernel,
        out_shape=jax.ShapeDtypeStruct((Bp, D), jnp.float32),
        grid=(nt,),
        in_specs=[
            pl.BlockSpec((1, 1, TB), lambda t: (t, 0, 0),
                         memory_space=pltpu.SMEM),
            pl.BlockSpec((Nu, 1, D), lambda t: (0, 0, 0)),
        ],
        out_specs=pl.BlockSpec((TB, D), lambda t: (t, 0)),
        compiler_params=cparams,
    )(u_ids, ut)

    out = pl.pallas_call(
        _item_compute_kernel,
        out_shape=jax.ShapeDtypeStruct((3, Bp), jnp.float32),
        grid=(nt,),
        in_specs=[
            pl.BlockSpec((1, 2, TB), lambda t: (t, 0, 0),
                         memory_space=pltpu.SMEM),
            pl.BlockSpec((Ni, 1, 2 * D), lambda t: (0, 0, 0)),
            pl.BlockSpec((TB, D), lambda t: (t, 0)),
        ],
        out_specs=pl.BlockSpec((3, TB), lambda t: (0, t)),
        scratch_shapes=[
            pltpu.VMEM((TB, 2 * D), jnp.float32),
            pltpu.VMEM((TB, 2 * D), jnp.float32),
        ],
        compiler_params=cparams2,
    )(ij_ids, it_aug, ug)

    pred_i = out[0, :B]
    pred_j = out[1, :B]
    reg_loss = 0.5 * jnp.sum(out[2, :B]) / B
    return pred_i, pred_j, reg_loss


# speedup vs baseline: 1.0820x; 1.0820x over previous
import jax
import jax.numpy as jnp
from jax import lax
from jax.experimental import pallas as pl
from jax.experimental.pallas import tpu as pltpu

_TB = 1024
_CB = 2000


def _softplus(x):
    return jnp.logaddexp(x, 0.0)


def _itaug_build_kernel(emb_ref, g_ref, out_ref):
    CB, _, D = emb_ref.shape
    out_ref[:, :, :D] = emb_ref[...]
    out_ref[:, :, D:D + 1] = g_ref[...].reshape(CB, 1, 1)
    out_ref[:, :, D + 1:] = jnp.zeros((CB, 1, D - 1), jnp.float32)


def _user_gather_kernel(ids_ref, tab_ref, out_ref):
    for mi in range(out_ref.shape[0]):
        out_ref[mi] = tab_ref[ids_ref[0, 0, mi], 0]


def _item_compute_kernel(ids_ref, tab_ref, u_ref, out_ref, vi_s, vj_s):
    TB, D = u_ref.shape
    K = 3 * D + 4

    for mi in range(TB):
        vi_s[mi] = tab_ref[ids_ref[0, 0, mi], 0]
        vj_s[mi] = tab_ref[ids_ref[0, 1, mi], 0]

    u = u_ref[...]
    vi = vi_s[:, :D]
    vj = vj_s[:, :D]
    gi = vi_s[:, D:D + 1]
    gj = vj_s[:, D:D + 1]

    gi_h = gi.astype(jnp.bfloat16).astype(jnp.float32)
    gj_h = gj.astype(jnp.bfloat16).astype(jnp.float32)
    gi_l = gi - gi_h
    gj_l = gj - gj_h

    P = jnp.concatenate(
        [u * vi, u * vj, u * u + vi * vi + vj * vj,
         gi_h, gj_h, gi_l, gj_l], axis=1)
    rowi = lax.broadcasted_iota(jnp.int32, (8, K), 0)
    coli = lax.broadcasted_iota(jnp.int32, (8, K), 1)
    S = (((rowi == coli // D) & (coli < 3 * D))
         | ((coli >= 3 * D) & (rowi == coli - 3 * D + 3))).astype(jnp.float32)
    R = lax.dot_general(S, P, (((1,), (1,)), ((), ())),
                        preferred_element_type=jnp.float32)

    gate = R[3:5, :] + R[5:7, :]
    pred = _softplus(R[0:2, :]) * gate
    out_ref[...] = jnp.concatenate([pred, R[2:3, :]], axis=0)


def kernel(embed_user, embed_item, q, user, item_i, item_j):
    B = int(user.shape[0])
    Nu, D = int(embed_user.shape[0]), int(embed_user.shape[1])
    Ni = int(embed_item.shape[0])

    TB = _TB
    nt = -(-B // TB)
    Bp = nt * TB
    pad = Bp - B

    def pad_ids(x):
        x = x.astype(jnp.int32)
        return jnp.pad(x, (0, pad)) if pad else x

    u_ids = pad_ids(user)
    ij_ids = jnp.stack([pad_ids(item_i).reshape(nt, TB),
                        pad_ids(item_j).reshape(nt, TB)], axis=1)

    cparams = pltpu.CompilerParams(
        dimension_semantics=("arbitrary",),
        vmem_limit_bytes=57 * 1024 * 1024,
    )
    cparams2 = pltpu.CompilerParams(
        dimension_semantics=("arbitrary",),
        vmem_limit_bytes=63 * 1024 * 1024,
    )

    CB = _CB
    nb = -(-Ni // CB)
    emb3 = embed_item.astype(jnp.float32).reshape(Ni, 1, D)
    g = jnp.tanh(_softplus(q.astype(jnp.float32))).reshape(Ni, 1)
    it_aug = pl.pallas_call(
        _itaug_build_kernel,
        out_shape=jax.ShapeDtypeStruct((Ni, 1, 2 * D), jnp.float32),
        grid=(nb,),
        in_specs=[
            pl.BlockSpec((CB, 1, D), lambda t: (t, 0, 0)),
            pl.BlockSpec((CB, 1), lambda t: (t, 0)),
        ],
        out_specs=pl.BlockSpec((CB, 1, 2 * D), lambda t: (t, 0, 0)),
        compiler_params=pltpu.CompilerParams(
            dimension_semantics=("arbitrary",),
            vmem_limit_bytes=32 * 1024 * 1024,
        ),
    )(emb3, g)

    ug = jnp.take(embed_user.astype(jnp.float32), u_ids, axis=0)

    out = pl.pallas_call(
        _item_compute_kernel,
        out_shape=jax.ShapeDtypeStruct((3, Bp), jnp.float32),
        grid=(nt,),
        in_specs=[
            pl.BlockSpec((1, 2, TB), lambda t: (t, 0, 0),
                         memory_space=pltpu.SMEM),
            pl.BlockSpec((Ni, 1, 2 * D), lambda t: (0, 0, 0)),
            pl.BlockSpec((TB, D), lambda t: (t, 0)),
        ],
        out_specs=pl.BlockSpec((3, TB), lambda t: (0, t)),
        scratch_shapes=[
            pltpu.VMEM((TB, 2 * D), jnp.float32),
            pltpu.VMEM((TB, 2 * D), jnp.float32),
        ],
        compiler_params=cparams2,
    )(ij_ids, it_aug, ug)

    pred_i = out[0, :B]
    pred_j = out[1, :B]
    reg_loss = 0.5 * jnp.sum(out[2, :B]) / B
    return pred_i, pred_j, reg_loss


# speedup vs baseline: 1.1731x; 1.0842x over previous
import jax
import jax.numpy as jnp
from jax import lax
from jax.experimental import pallas as pl
from jax.experimental.pallas import tpu as pltpu

_TB = 1024
_CB = 2000


def _softplus(x):
    return jnp.logaddexp(x, 0.0)


def _itaug_build_kernel(emb_ref, g_ref, out_ref):
    CB, _, D = emb_ref.shape
    out_ref[:, :, :D] = emb_ref[...]
    out_ref[:, :, D:D + 1] = g_ref[...].reshape(CB, 1, 1)
    out_ref[:, :, D + 1:] = jnp.zeros((CB, 1, D - 1), jnp.float32)


def _item_compute_kernel(ids_ref, tab_ref, u_ref, out_ref, vi_s, vj_s):
    TB, D = u_ref.shape
    K = 3 * D + 4

    for mi in range(TB):
        vi_s[mi] = tab_ref[ids_ref[0, 0, mi], 0]
        vj_s[mi] = tab_ref[ids_ref[0, 1, mi], 0]

    u = u_ref[...]
    vi = vi_s[:, :D]
    vj = vj_s[:, :D]
    gi = vi_s[:, D:D + 1]
    gj = vj_s[:, D:D + 1]

    gi_h = gi.astype(jnp.bfloat16).astype(jnp.float32)
    gj_h = gj.astype(jnp.bfloat16).astype(jnp.float32)
    gi_l = gi - gi_h
    gj_l = gj - gj_h

    P = jnp.concatenate(
        [u * vi, u * vj, u * u + vi * vi + vj * vj,
         gi_h, gj_h, gi_l, gj_l], axis=1)
    rowi = lax.broadcasted_iota(jnp.int32, (8, K), 0)
    coli = lax.broadcasted_iota(jnp.int32, (8, K), 1)
    S = (((rowi == coli // D) & (coli < 3 * D))
         | ((coli >= 3 * D) & (rowi == coli - 3 * D + 3))).astype(jnp.float32)
    R = lax.dot_general(S, P, (((1,), (1,)), ((), ())),
                        preferred_element_type=jnp.float32)

    gate = R[3:5, :] + R[5:7, :]
    pred = _softplus(R[0:2, :]) * gate
    out_ref[...] = jnp.concatenate([pred, R[2:3, :]], axis=0)


def kernel(embed_user, embed_item, q, user, item_i, item_j):
    B = int(user.shape[0])
    Nu, D = int(embed_user.shape[0]), int(embed_user.shape[1])
    Ni = int(embed_item.shape[0])

    TB = _TB
    nt = -(-B // TB)
    Bp = nt * TB
    pad = Bp - B

    def pad_ids(x):
        x = x.astype(jnp.int32)
        return jnp.pad(x, (0, pad)) if pad else x

    u_ids = pad_ids(user)
    ij_ids = jnp.stack([pad_ids(item_i).reshape(nt, TB),
                        pad_ids(item_j).reshape(nt, TB)], axis=1)

    cparams2 = pltpu.CompilerParams(
        dimension_semantics=("arbitrary",),
        vmem_limit_bytes=63 * 1024 * 1024,
    )

    CB = _CB
    nb = -(-Ni // CB)
    emb3 = embed_item.astype(jnp.float32).reshape(Ni, 1, D)
    g = jnp.tanh(_softplus(q.astype(jnp.float32))).reshape(Ni, 1)
    it_aug = pl.pallas_call(
        _itaug_build_kernel,
        out_shape=jax.ShapeDtypeStruct((Ni, 1, 2 * D), jnp.float32),
        grid=(nb,),
        in_specs=[
            pl.BlockSpec((CB, 1, D), lambda t: (t, 0, 0)),
            pl.BlockSpec((CB, 1), lambda t: (t, 0)),
        ],
        out_specs=pl.BlockSpec((CB, 1, 2 * D), lambda t: (t, 0, 0)),
        compiler_params=pltpu.CompilerParams(
            dimension_semantics=("arbitrary",),
            vmem_limit_bytes=32 * 1024 * 1024,
        ),
    )(emb3, g)

    ug = embed_user.at[u_ids].get(mode="promise_in_bounds")

    out = pl.pallas_call(
        _item_compute_kernel,
        out_shape=jax.ShapeDtypeStruct((3, Bp), jnp.float32),
        grid=(nt,),
        in_specs=[
            pl.BlockSpec((1, 2, TB), lambda t: (t, 0, 0),
                         memory_space=pltpu.SMEM),
            pl.BlockSpec((Ni, 1, 2 * D), lambda t: (0, 0, 0)),
            pl.BlockSpec((TB, D), lambda t: (t, 0)),
        ],
        out_specs=pl.BlockSpec((3, TB), lambda t: (0, t)),
        scratch_shapes=[
            pltpu.VMEM((TB, 2 * D), jnp.float32),
            pltpu.VMEM((TB, 2 * D), jnp.float32),
        ],
        compiler_params=cparams2,
    )(ij_ids, it_aug, ug)

    pred_i = out[0, :B]
    pred_j = out[1, :B]
    reg_loss = 0.5 * jnp.sum(out[2, :B]) / B
    return pred_i, pred_j, reg_loss
